# baseline (speedup 1.0000x reference)
import functools

import jax
import jax.numpy as jnp
from jax import lax
from jax.experimental import pallas as pl
from jax.experimental.pallas import tpu as pltpu
from jax.experimental.pallas import tpu_sc as plsc

D = 64
N_S = 883
N_T = 12
B = 1024
EPS = 1e-5

NW = 32
B_SC = 256
B_PER_W = B // NW

ROWS_W = B_SC * N_S // NW
S_ROWS = 192
CH_FULL = ROWS_W // S_ROWS
CH_TAIL = ROWS_W - CH_FULL * S_ROWS

T_BB = 4


def _rsqrt_nr(a):
    i = lax.bitcast_convert_type(a, jnp.int32)
    i = jnp.int32(0x5F3759DF) - lax.shift_right_arithmetic(i, 1)
    y = lax.bitcast_convert_type(i, jnp.float32)
    for _ in range(3):
        y = y * (1.5 - 0.5 * a * y * y)
    return y


def _ln_rows(buf, rows, tab, trows, g, bt):
    n = len(rows)
    ys = [None] * n
    rs = [None] * n
    for r in range(n):
        y = []
        for j in range(4):
            x = buf[(*rows[r], pl.ds(j * 16, 16))]
            t = tab[pl.ds(trows[r] + j * 16, 16)]
            y.append(x + t)
        ys[r] = y
    for r in range(n):
        y = ys[r]
        s = (y[0] + y[1]) + (y[2] + y[3])
        q = (y[0] * y[0] + y[1] * y[1]) + (y[2] * y[2] + y[3] * y[3])
        sv = jnp.broadcast_to(jnp.sum(s), (16,))
        qv = jnp.broadcast_to(jnp.sum(q), (16,))
        mean = sv * (1.0 / 64.0)
        var = qv * (1.0 / 64.0) - mean * mean
        rs[r] = (mean, _rsqrt_nr(var + EPS))
    for r in range(n):
        y = ys[r]
        mean, rinv = rs[r]
        for j in range(4):
            out = (y[j] - mean) * (rinv * g[j]) + bt[j]
            buf[(*rows[r], pl.ds(j * 16, 16))] = out


def _compute_s_chunk(buf, tabbuf, c, t0, gs, bs):
    @plsc.parallel_loop(0, c // 4, unroll=2)
    def rows4(i):
        r0 = i * 4
        trows = []
        for r in range(4):
            tr = t0 + r0 + r
            trows.append(jnp.where(tr >= N_S, tr - N_S, tr) * D)
        _ln_rows(buf, [(r0 + r,) for r in range(4)], tabbuf, trows, gs, bs)


def _sc_body(s_x, t_x, tab_s, tab_t, g_s, b_s, g_t, b_t,
             s_out, t_out,
             xbufa, xbufb, tabbuf, tbuf, ttabbuf, gbbuf,
             sem_ain, sem_aout, sem_bin, sem_bout):
    wid = lax.axis_index("s") * 2 + lax.axis_index("c")
    bt0 = wid * B_PER_W

    pltpu.sync_copy(g_s, gbbuf.at[pl.ds(0, D)])
    pltpu.sync_copy(b_s, gbbuf.at[pl.ds(D, D)])
    pltpu.sync_copy(g_t, gbbuf.at[pl.ds(2 * D, D)])
    pltpu.sync_copy(b_t, gbbuf.at[pl.ds(3 * D, D)])
    gs = [gbbuf[pl.ds(j * 16, 16)] for j in range(4)]
    bs = [gbbuf[pl.ds(D + j * 16, 16)] for j in range(4)]
    gt = [gbbuf[pl.ds(2 * D + j * 16, 16)] for j in range(4)]
    bt = [gbbuf[pl.ds(3 * D + j * 16, 16)] for j in range(4)]

    pltpu.sync_copy(tab_s, tabbuf)
    r0w = wid * ROWS_W

    def in_cp(k, size, buf, sem):
        off = pl.multiple_of(r0w + k * S_ROWS, 8)
        return pltpu.make_async_copy(
            s_x.at[pl.ds(off, size)], buf.at[pl.ds(0, size)], sem)

    def out_cp(k, size, buf, sem):
        off = pl.multiple_of(r0w + k * S_ROWS, 8)
        return pltpu.make_async_copy(
            buf.at[pl.ds(0, size)], s_out.at[pl.ds(off, size)], sem)

    def t0_of(k):
        return lax.rem(k * S_ROWS, N_S)

    in_cp(0, S_ROWS, xbufa, sem_ain).start()
    n_pairs = CH_FULL // 2

    def pair(i, _):
        ka, kb = 2 * i, 2 * i + 1

        @pl.when(i > 0)
        def _():
            out_cp(kb, S_ROWS, xbufb, sem_bout).wait()

        in_cp(kb, S_ROWS, xbufb, sem_bin).start()
        in_cp(ka, S_ROWS, xbufa, sem_ain).wait()
        _compute_s_chunk(xbufa, tabbuf, S_ROWS, t0_of(ka), gs, bs)
        out_cp(ka, S_ROWS, xbufa, sem_aout).start()
        in_cp(kb, S_ROWS, xbufb, sem_bin).wait()
        _compute_s_chunk(xbufb, tabbuf, S_ROWS, t0_of(kb), gs, bs)
        out_cp(kb, S_ROWS, xbufb, sem_bout).start()

        @pl.when(i < n_pairs - 1)
        def _():
            out_cp(ka, S_ROWS, xbufa, sem_aout).wait()
            in_cp(ka + 2, S_ROWS, xbufa, sem_ain).start()

        return 0

    lax.fori_loop(0, n_pairs, pair, 0)
    kt = CH_FULL
    out_cp(kt - 2, S_ROWS, xbufa, sem_aout).wait()
    in_cp(kt, CH_TAIL, xbufa, sem_ain).start()
    in_cp(kt, CH_TAIL, xbufa, sem_ain).wait()
    _compute_s_chunk(xbufa, tabbuf, CH_TAIL, (kt * S_ROWS) % N_S, gs, bs)
    out_cp(kt, CH_TAIL, xbufa, sem_aout).start()
    out_cp(kt, CH_TAIL, xbufa, sem_aout).wait()
    out_cp(kt - 1, S_ROWS, xbufb, sem_bout).wait()

    pltpu.sync_copy(tab_t, ttabbuf)

    def t_chunk(ci, _):
        tb = pl.multiple_of((bt0 + ci * T_BB) * N_T, 8)
        pltpu.sync_copy(t_x.at[pl.ds(tb, T_BB * N_T)], tbuf)

        @plsc.parallel_loop(0, T_BB)
        def t_batch(q):
            for half in range(2):
                _ln_rows(tbuf, [(q * N_T + half * 6 + v,) for v in range(6)],
                         ttabbuf, [(half * 6 + v) * D for v in range(6)],
                         gt, bt)

        pltpu.sync_copy(tbuf, t_out.at[pl.ds(tb, T_BB * N_T)])
        return 0

    lax.fori_loop(0, B_PER_W // T_BB, t_chunk, 0)


BB = 4


def _tc_ln_body(x_ref, tab_ref, g_ref, b_ref, o_ref):
    y = x_ref[...] + tab_ref[...][None]
    ones = jnp.ones((D, 1), jnp.float32)
    s1 = lax.dot_general(y, ones, (((2,), (0,)), ((), ())),
                         preferred_element_type=jnp.float32)
    s2 = lax.dot_general(y * y, ones, (((2,), (0,)), ((), ())),
                         preferred_element_type=jnp.float32)
    mean = s1 * (1.0 / 64.0)
    var = s2 * (1.0 / 64.0) - mean * mean
    o_ref[...] = (y - mean) * (lax.rsqrt(var + EPS) * g_ref[...]) + b_ref[...]


def _tc_call(s_x, tab_s, g_s, b_s):
    return pl.pallas_call(
        _tc_ln_body,
        grid=((B - B_SC) // BB,),
        in_specs=[
            pl.BlockSpec((BB, N_S, D), lambda b: (b + B_SC // BB, 0, 0)),
            pl.BlockSpec((N_S, D), lambda b: (0, 0)),
            pl.BlockSpec((D,), lambda b: (0,)),
            pl.BlockSpec((D,), lambda b: (0,)),
        ],
        out_specs=pl.BlockSpec((BB, N_S, D), lambda b: (b + B_SC // BB, 0, 0)),
        out_shape=jax.ShapeDtypeStruct((B, N_S, D), jnp.float32),
        compiler_params=pltpu.CompilerParams(
            dimension_semantics=("arbitrary",)),
    )(s_x, tab_s, g_s, b_s)


def _stitch_body(full_ref, sc_ref, o_ref):
    o_ref[...] = sc_ref[...].reshape(8, N_S, D)
    del full_ref


def _stitch(tc_s, sc_s):
    return pl.pallas_call(
        _stitch_body,
        grid=(B_SC // 8,),
        in_specs=[
            pl.BlockSpec((8, 8, D), lambda b: (0, 0, 0)),
            pl.BlockSpec((8 * N_S, D), lambda b: (b, 0)),
        ],
        out_specs=pl.BlockSpec((8, N_S, D), lambda b: (b, 0, 0)),
        out_shape=jax.ShapeDtypeStruct((B, N_S, D), jnp.float32),
        input_output_aliases={0: 0},
        compiler_params=pltpu.CompilerParams(
            dimension_semantics=("arbitrary",)),
    )(tc_s, sc_s)


@jax.jit
def _run(s_x, t_x, tab_s, tab_t, g_s, b_s, g_t, b_t):
    mesh = plsc.VectorSubcoreMesh(core_axis_name="c", subcore_axis_name="s")
    kern = pl.kernel(
        _sc_body,
        out_type=[
            jax.ShapeDtypeStruct((B_SC * N_S, D), jnp.float32),
            jax.ShapeDtypeStruct((B * N_T, D), jnp.float32),
        ],
        mesh=mesh,
        compiler_params=pltpu.CompilerParams(needs_layout_passes=False),
        scratch_types=[
            pltpu.VMEM((S_ROWS, D), jnp.float32),
            pltpu.VMEM((S_ROWS, D), jnp.float32),
            pltpu.VMEM((N_S * D,), jnp.float32),
            pltpu.VMEM((T_BB * N_T, D), jnp.float32),
            pltpu.VMEM((N_T * D,), jnp.float32),
            pltpu.VMEM((4 * D,), jnp.float32),
            pltpu.SemaphoreType.DMA,
            pltpu.SemaphoreType.DMA,
            pltpu.SemaphoreType.DMA,
            pltpu.SemaphoreType.DMA,
        ],
    )
    sc_s, t_out = kern(
        s_x.reshape(B * N_S, D), t_x.reshape(B * N_T, D),
        tab_s.reshape(-1), tab_t.reshape(-1), g_s, b_s, g_t, b_t)
    tc_s = _tc_call(s_x, tab_s, g_s, b_s)
    s_out = _stitch(tc_s, sc_s)
    return s_out, t_out.reshape(B, N_T, D)


def kernel(s_x, t_x, pos_s_table, pos_t_table, gamma_s, beta_s, gamma_t, beta_t):
    return tuple(_run(s_x, t_x, pos_s_table, pos_t_table,
                      gamma_s, beta_s, gamma_t, beta_t))


# speedup vs baseline: 2.2501x; 2.2501x over previous
import jax
import jax.numpy as jnp
from jax import lax
from jax.experimental import pallas as pl
from jax.experimental.pallas import tpu as pltpu
from jax.experimental.pallas import tpu_sc as plsc

D = 64
N_S = 883
N_T = 12
B = 1024
EPS = 1e-5

NW = 32
B_PER_W = B // NW
T_BB = 4

TC_ROWS = B * N_S * D // 128
TC_BLK = 7064
TC_GRID = TC_ROWS // TC_BLK


def _rsqrt_nr(a):
    i = lax.bitcast_convert_type(a, jnp.int32)
    i = jnp.int32(0x5F3759DF) - lax.shift_right_arithmetic(i, 1)
    y = lax.bitcast_convert_type(i, jnp.float32)
    for _ in range(3):
        y = y * (1.5 - 0.5 * a * y * y)
    return y


def _ln_rows(buf, rows, tab, trows, g, bt):
    n = len(rows)
    ys = [None] * n
    rs = [None] * n
    for r in range(n):
        ys[r] = [buf[(*rows[r], pl.ds(j * 16, 16))] +
                 tab[pl.ds(trows[r] + j * 16, 16)] for j in range(4)]
    for r in range(n):
        y = ys[r]
        s = (y[0] + y[1]) + (y[2] + y[3])
        q = (y[0] * y[0] + y[1] * y[1]) + (y[2] * y[2] + y[3] * y[3])
        sv = jnp.broadcast_to(jnp.sum(s), (16,))
        qv = jnp.broadcast_to(jnp.sum(q), (16,))
        mean = sv * (1.0 / 64.0)
        var = qv * (1.0 / 64.0) - mean * mean
        rs[r] = (mean, _rsqrt_nr(var + EPS))
    for r in range(n):
        y = ys[r]
        mean, rinv = rs[r]
        for j in range(4):
            out = (y[j] - mean) * (rinv * g[j]) + bt[j]
            buf[(*rows[r], pl.ds(j * 16, 16))] = out


def _sc_body(t_x, tab_t, g_t, b_t, t_out, tbuf, ttabbuf, gbbuf):
    wid = lax.axis_index("s") * 2 + lax.axis_index("c")
    bt0 = wid * B_PER_W

    pltpu.sync_copy(g_t, gbbuf.at[pl.ds(0, D)])
    pltpu.sync_copy(b_t, gbbuf.at[pl.ds(D, D)])
    gt = [gbbuf[pl.ds(j * 16, 16)] for j in range(4)]
    bt = [gbbuf[pl.ds(D + j * 16, 16)] for j in range(4)]
    pltpu.sync_copy(tab_t, ttabbuf)

    def t_chunk(ci, _):
        tb = pl.multiple_of((bt0 + ci * T_BB) * N_T, 8)
        pltpu.sync_copy(t_x.at[pl.ds(tb, T_BB * N_T)], tbuf)

        @plsc.parallel_loop(0, T_BB)
        def t_batch(q):
            for half in range(2):
                _ln_rows(tbuf, [(q * N_T + half * 6 + v,) for v in range(6)],
                         ttabbuf, [(half * 6 + v) * D for v in range(6)],
                         gt, bt)

        pltpu.sync_copy(tbuf, t_out.at[pl.ds(tb, T_BB * N_T)])
        return 0

    lax.fori_loop(0, B_PER_W // T_BB, t_chunk, 0)


def _tc_ln_body(x_ref, tab_ref, g_ref, b_ref, o_ref):
    y = x_ref[...] + tab_ref[...]
    i = lax.broadcasted_iota(jnp.int32, (128, 128), 0)
    j = lax.broadcasted_iota(jnp.int32, (128, 128), 1)
    m = jnp.where((i < 64) == (j < 64), 1.0, 0.0)
    s1 = lax.dot_general(y, m, (((1,), (0,)), ((), ())),
                         preferred_element_type=jnp.float32)
    s2 = lax.dot_general(y * y, m, (((1,), (0,)), ((), ())),
                         preferred_element_type=jnp.float32)
    mean = s1 * (1.0 / 64.0)
    var = s2 * (1.0 / 64.0) - mean * mean
    o_ref[...] = (y - mean) * (lax.rsqrt(var + EPS) * g_ref[...]) + b_ref[...]


def _tc_call(s_x2, tab16, g2, b2):
    return pl.pallas_call(
        _tc_ln_body,
        grid=(TC_GRID,),
        in_specs=[
            pl.BlockSpec((TC_BLK, 128), lambda b: (b, 0)),
            pl.BlockSpec((TC_BLK, 128), lambda b: (0, 0)),
            pl.BlockSpec((128,), lambda b: (0,)),
            pl.BlockSpec((128,), lambda b: (0,)),
        ],
        out_specs=pl.BlockSpec((TC_BLK, 128), lambda b: (b, 0)),
        out_shape=jax.ShapeDtypeStruct((TC_ROWS, 128), jnp.float32),
        compiler_params=pltpu.CompilerParams(
            dimension_semantics=("arbitrary",)),
    )(s_x2, tab16, g2, b2)


@jax.jit
def _run(s_x, t_x, tab_s, tab_t, g_s, b_s, g_t, b_t):
    mesh = plsc.VectorSubcoreMesh(core_axis_name="c", subcore_axis_name="s")
    kern = pl.kernel(
        _sc_body,
        out_type=jax.ShapeDtypeStruct((B * N_T, D), jnp.float32),
        mesh=mesh,
        compiler_params=pltpu.CompilerParams(needs_layout_passes=False),
        scratch_types=[
            pltpu.VMEM((T_BB * N_T, D), jnp.float32),
            pltpu.VMEM((N_T * D,), jnp.float32),
            pltpu.VMEM((2 * D,), jnp.float32),
        ],
    )
    t_out = kern(t_x.reshape(B * N_T, D), tab_t.reshape(-1), g_t, b_t)

    s_x2 = s_x.reshape(TC_ROWS, 128)
    tab16 = jnp.tile(tab_s.reshape(-1), 16).reshape(TC_BLK, 128)
    g2 = jnp.concatenate([g_s, g_s])
    b2 = jnp.concatenate([b_s, b_s])
    tc_s = _tc_call(s_x2, tab16, g2, b2)
    return tc_s.reshape(B, N_S, D), t_out.reshape(B, N_T, D)


def kernel(s_x, t_x, pos_s_table, pos_t_table, gamma_s, beta_s, gamma_t, beta_t):
    return tuple(_run(s_x, t_x, pos_s_table, pos_t_table,
                      gamma_s, beta_s, gamma_t, beta_t))


# speedup vs baseline: 2.3130x; 1.0279x over previous
import jax
import jax.numpy as jnp
from jax import lax
from jax.experimental import pallas as pl
from jax.experimental.pallas import tpu as pltpu
from jax.experimental.pallas import tpu_sc as plsc

D = 64
N_S = 883
N_T = 12
B = 1024
EPS = 1e-5

NW = 32
B_SC = 256
B_PER_W = B // NW

ROWS_W = B_SC * N_S // NW
S_ROWS = 192
CH_FULL = ROWS_W // S_ROWS
CH_TAIL = ROWS_W - CH_FULL * S_ROWS

T_BB = 4


def _rsqrt_nr(a):
    i = lax.bitcast_convert_type(a, jnp.int32)
    i = jnp.int32(0x5F3759DF) - lax.shift_right_arithmetic(i, 1)
    y = lax.bitcast_convert_type(i, jnp.float32)
    for _ in range(3):
        y = y * (1.5 - 0.5 * a * y * y)
    return y


def _ln_rows(buf, rows, tab, trows, g, bt):
    n = len(rows)
    ys = [None] * n
    rs = [None] * n
    for r in range(n):
        ys[r] = [buf[(*rows[r], pl.ds(j * 16, 16))] +
                 tab[pl.ds(trows[r] + j * 16, 16)] for j in range(4)]
    for r in range(n):
        y = ys[r]
        s = (y[0] + y[1]) + (y[2] + y[3])
        q = (y[0] * y[0] + y[1] * y[1]) + (y[2] * y[2] + y[3] * y[3])
        sv = jnp.broadcast_to(jnp.sum(s), (16,))
        qv = jnp.broadcast_to(jnp.sum(q), (16,))
        mean = sv * (1.0 / 64.0)
        var = qv * (1.0 / 64.0) - mean * mean
        rs[r] = (mean, _rsqrt_nr(var + EPS))
    for r in range(n):
        y = ys[r]
        mean, rinv = rs[r]
        for j in range(4):
            out = (y[j] - mean) * (rinv * g[j]) + bt[j]
            buf[(*rows[r], pl.ds(j * 16, 16))] = out


def _compute_s_chunk(buf, tabbuf, c, t0, gs, bs):
    @plsc.parallel_loop(0, c // 4, unroll=2)
    def rows4(i):
        r0 = i * 4
        trows = []
        for r in range(4):
            tr = t0 + r0 + r
            trows.append(jnp.where(tr >= N_S, tr - N_S, tr) * D)
        _ln_rows(buf, [(r0 + r,) for r in range(4)], tabbuf, trows, gs, bs)


def _sc_body(s_x, t_x, tab_s, tab_t, g_s, b_s, g_t, b_t,
             s_out, t_out,
             xbufa, xbufb, tabbuf, tbuf, ttabbuf, gbbuf,
             sem_ain, sem_aout, sem_bin, sem_bout):
    wid = lax.axis_index("s") * 2 + lax.axis_index("c")
    bt0 = wid * B_PER_W

    pltpu.sync_copy(g_s, gbbuf.at[pl.ds(0, D)])
    pltpu.sync_copy(b_s, gbbuf.at[pl.ds(D, D)])
    pltpu.sync_copy(g_t, gbbuf.at[pl.ds(2 * D, D)])
    pltpu.sync_copy(b_t, gbbuf.at[pl.ds(3 * D, D)])
    gs = [gbbuf[pl.ds(j * 16, 16)] for j in range(4)]
    bs = [gbbuf[pl.ds(D + j * 16, 16)] for j in range(4)]
    gt = [gbbuf[pl.ds(2 * D + j * 16, 16)] for j in range(4)]
    bt = [gbbuf[pl.ds(3 * D + j * 16, 16)] for j in range(4)]

    pltpu.sync_copy(tab_s, tabbuf)
    r0w = wid * ROWS_W

    def in_cp(k, size, buf, sem):
        off = pl.multiple_of(r0w + k * S_ROWS, 8)
        return pltpu.make_async_copy(
            s_x.at[pl.ds(off, size)], buf.at[pl.ds(0, size)], sem)

    def out_cp(k, size, buf, sem):
        off = pl.multiple_of(r0w + k * S_ROWS, 8)
        return pltpu.make_async_copy(
            buf.at[pl.ds(0, size)], s_out.at[pl.ds(off, size)], sem)

    def t0_of(k):
        return lax.rem(k * S_ROWS, N_S)

    in_cp(0, S_ROWS, xbufa, sem_ain).start()
    n_pairs = CH_FULL // 2

    def pair(i, _):
        ka, kb = 2 * i, 2 * i + 1

        @pl.when(i > 0)
        def _():
            out_cp(kb, S_ROWS, xbufb, sem_bout).wait()

        in_cp(kb, S_ROWS, xbufb, sem_bin).start()
        in_cp(ka, S_ROWS, xbufa, sem_ain).wait()
        _compute_s_chunk(xbufa, tabbuf, S_ROWS, t0_of(ka), gs, bs)
        out_cp(ka, S_ROWS, xbufa, sem_aout).start()
        in_cp(kb, S_ROWS, xbufb, sem_bin).wait()
        _compute_s_chunk(xbufb, tabbuf, S_ROWS, t0_of(kb), gs, bs)
        out_cp(kb, S_ROWS, xbufb, sem_bout).start()

        @pl.when(i < n_pairs - 1)
        def _():
            out_cp(ka, S_ROWS, xbufa, sem_aout).wait()
            in_cp(ka + 2, S_ROWS, xbufa, sem_ain).start()

        return 0

    lax.fori_loop(0, n_pairs, pair, 0)
    kt = CH_FULL
    out_cp(kt - 2, S_ROWS, xbufa, sem_aout).wait()
    in_cp(kt, CH_TAIL, xbufa, sem_ain).start()
    in_cp(kt, CH_TAIL, xbufa, sem_ain).wait()
    _compute_s_chunk(xbufa, tabbuf, CH_TAIL, (kt * S_ROWS) % N_S, gs, bs)
    out_cp(kt, CH_TAIL, xbufa, sem_aout).start()
    out_cp(kt, CH_TAIL, xbufa, sem_aout).wait()
    out_cp(kt - 1, S_ROWS, xbufb, sem_bout).wait()

    pltpu.sync_copy(tab_t, ttabbuf)

    def t_chunk(ci, _):
        tb = pl.multiple_of((bt0 + ci * T_BB) * N_T, 8)
        pltpu.sync_copy(t_x.at[pl.ds(tb, T_BB * N_T)], tbuf)

        @plsc.parallel_loop(0, T_BB)
        def t_batch(q):
            for half in range(2):
                _ln_rows(tbuf, [(q * N_T + half * 6 + v,) for v in range(6)],
                         ttabbuf, [(half * 6 + v) * D for v in range(6)],
                         gt, bt)

        pltpu.sync_copy(tbuf, t_out.at[pl.ds(tb, T_BB * N_T)])
        return 0

    lax.fori_loop(0, B_PER_W // T_BB, t_chunk, 0)


BB = 4


def _tc_ln_body(x_ref, tab_ref, g_ref, b_ref, o_ref):
    y = x_ref[...] + tab_ref[...][None]
    mean = jnp.mean(y, axis=-1, keepdims=True)
    var = jnp.mean(y * y, axis=-1, keepdims=True) - mean * mean
    o_ref[...] = (y - mean) * (lax.rsqrt(var + EPS) * g_ref[...]) + b_ref[...]


def _tc_call(s_x, tab_s, g_s, b_s):
    return pl.pallas_call(
        _tc_ln_body,
        grid=((B - B_SC) // BB,),
        in_specs=[
            pl.BlockSpec((BB, N_S, D), lambda b: (b + B_SC // BB, 0, 0)),
            pl.BlockSpec((N_S, D), lambda b: (0, 0)),
            pl.BlockSpec((D,), lambda b: (0,)),
            pl.BlockSpec((D,), lambda b: (0,)),
        ],
        out_specs=pl.BlockSpec((BB, N_S, D), lambda b: (b + B_SC // BB, 0, 0)),
        out_shape=jax.ShapeDtypeStruct((B, N_S, D), jnp.float32),
        compiler_params=pltpu.CompilerParams(
            dimension_semantics=("arbitrary",)),
    )(s_x, tab_s, g_s, b_s)


@jax.jit
def _run(s_x, t_x, tab_s, tab_t, g_s, b_s, g_t, b_t):
    mesh = plsc.VectorSubcoreMesh(core_axis_name="c", subcore_axis_name="s")
    kern = pl.kernel(
        _sc_body,
        out_type=[
            jax.ShapeDtypeStruct((B_SC * N_S, D), jnp.float32),
            jax.ShapeDtypeStruct((B * N_T, D), jnp.float32),
        ],
        mesh=mesh,
        compiler_params=pltpu.CompilerParams(needs_layout_passes=False),
        scratch_types=[
            pltpu.VMEM((S_ROWS, D), jnp.float32),
            pltpu.VMEM((S_ROWS, D), jnp.float32),
            pltpu.VMEM((N_S * D,), jnp.float32),
            pltpu.VMEM((T_BB * N_T, D), jnp.float32),
            pltpu.VMEM((N_T * D,), jnp.float32),
            pltpu.VMEM((4 * D,), jnp.float32),
            pltpu.SemaphoreType.DMA,
            pltpu.SemaphoreType.DMA,
            pltpu.SemaphoreType.DMA,
            pltpu.SemaphoreType.DMA,
        ],
    )
    sc_s, t_out = kern(
        s_x.reshape(B * N_S, D), t_x.reshape(B * N_T, D),
        tab_s.reshape(-1), tab_t.reshape(-1), g_s, b_s, g_t, b_t)
    tc_s = _tc_call(s_x, tab_s, g_s, b_s)
    s_out = lax.dynamic_update_slice(tc_s, sc_s.reshape(B_SC, N_S, D),
                                     (0, 0, 0))
    return s_out, t_out.reshape(B, N_T, D)


def kernel(s_x, t_x, pos_s_table, pos_t_table, gamma_s, beta_s, gamma_t, beta_t):
    return tuple(_run(s_x, t_x, pos_s_table, pos_t_table,
                      gamma_s, beta_s, gamma_t, beta_t))
